# baseline (speedup 1.0000x reference)
import functools

import jax
import jax.numpy as jnp
from jax.experimental import pallas as pl

N = 10000
BI = 400
BX = 2000


def _s1_body(x_ref, w1_ref, s1_ref):
    s1_ref[...] = jnp.dot(x_ref[...], w1_ref[...],
                          preferred_element_type=jnp.float32)


def _layer1_body(adj_ref, s1_ref, b1_ref, w2_ref, s2_ref):
    h = jnp.dot(adj_ref[...], s1_ref[...],
                preferred_element_type=jnp.float32) + b1_ref[...]
    h = jnp.maximum(h, 0.0)
    s2_ref[...] = jnp.dot(h, w2_ref[...], preferred_element_type=jnp.float32)


def _layer2_body(adj_ref, s2_ref, b2_ref, out_ref):
    out_ref[...] = jnp.dot(adj_ref[...], s2_ref[...],
                           preferred_element_type=jnp.float32) + b2_ref[...]


@functools.partial(jax.jit, static_argnames=("interpret",))
def _gcn(x, adj, W1, b1, W2, b2, interpret=False):
    nfeat = x.shape[1]
    nhid = W1.shape[1]
    nclass = W2.shape[1]

    s1 = pl.pallas_call(
        _s1_body,
        grid=(N // BX,),
        in_specs=[
            pl.BlockSpec((BX, nfeat), lambda i: (i, 0)),
            pl.BlockSpec((nfeat, nhid), lambda i: (0, 0)),
        ],
        out_specs=pl.BlockSpec((BX, nhid), lambda i: (i, 0)),
        out_shape=jax.ShapeDtypeStruct((N, nhid), jnp.float32),
        interpret=interpret,
    )(x, W1)

    s2 = pl.pallas_call(
        _layer1_body,
        grid=(N // BI,),
        in_specs=[
            pl.BlockSpec((BI, N), lambda i: (i, 0)),
            pl.BlockSpec((N, nhid), lambda i: (0, 0)),
            pl.BlockSpec((1, nhid), lambda i: (0, 0)),
            pl.BlockSpec((nhid, nclass), lambda i: (0, 0)),
        ],
        out_specs=pl.BlockSpec((BI, nclass), lambda i: (i, 0)),
        out_shape=jax.ShapeDtypeStruct((N, nclass), jnp.float32),
        interpret=interpret,
    )(adj, s1, b1.reshape(1, -1), W2)

    return pl.pallas_call(
        _layer2_body,
        grid=(N // BI,),
        in_specs=[
            pl.BlockSpec((BI, N), lambda i: (i, 0)),
            pl.BlockSpec((N, nclass), lambda i: (0, 0)),
            pl.BlockSpec((1, nclass), lambda i: (0, 0)),
        ],
        out_specs=pl.BlockSpec((BI, nclass), lambda i: (i, 0)),
        out_shape=jax.ShapeDtypeStruct((N, nclass), jnp.float32),
        interpret=interpret,
    )(adj, s2, b2.reshape(1, -1))


def kernel(x, adj, W1, b1, W2, b2):
    return _gcn(x, adj, W1, b1, W2, b2)
